# baseline (speedup 1.0000x reference)
import functools

import jax
import jax.numpy as jnp
from jax import lax
from jax.experimental import pallas as pl
from jax.experimental.pallas import tpu as pltpu
from jax.experimental.pallas import tpu_sc as plsc

_L = 16
_CHUNK = 800
_NSUB = 16
_NSC = 2


def _face_normals_impl(B, V, F):
    rounds = B // _NSC
    n_chunks = F // _CHUNK
    chunk_iters = -(-n_chunks // _NSUB)
    groups = _CHUNK // _L
    C = _CHUNK

    mesh = plsc.VectorSubcoreMesh(core_axis_name="c", subcore_axis_name="s")

    @functools.partial(
        pl.kernel,
        mesh=mesh,
        out_type=jax.ShapeDtypeStruct((3 * B * F,), jnp.float32),
        scratch_types=[
            pltpu.VMEM((C,), jnp.int32),
            pltpu.VMEM((C,), jnp.int32),
            pltpu.VMEM((C,), jnp.int32),
            pltpu.VMEM((C,), jnp.float32),
            pltpu.VMEM((C,), jnp.float32),
            pltpu.VMEM((C,), jnp.float32),
            pltpu.VMEM((C,), jnp.float32),
            pltpu.VMEM((C,), jnp.float32),
            pltpu.VMEM((C,), jnp.float32),
            pltpu.VMEM((C,), jnp.float32),
            pltpu.VMEM((C,), jnp.float32),
            pltpu.VMEM((C,), jnp.float32),
            pltpu.VMEM((C,), jnp.float32),
            pltpu.VMEM((C,), jnp.float32),
            pltpu.VMEM((C,), jnp.float32),
            pltpu.VMEM((V,), jnp.float32),
            pltpu.VMEM_SHARED((V,), jnp.float32),
            pltpu.VMEM_SHARED((V,), jnp.float32),
            pltpu.VMEM_SHARED((V,), jnp.float32),
            pltpu.SemaphoreType.DMA,
        ],
        compiler_params=pltpu.CompilerParams(needs_layout_passes=False),
    )
    def body(xp, yp, zp, fc_hbm, out_hbm,
             i0, i1, i2, x0b, y0b, z0b, x1b, y1b, z1b, x2b, y2b, z2b,
             oxb, oyb, ozb, stg, xs, ys, zs, sem):
        sc = lax.axis_index("c")
        sub = lax.axis_index("s")
        BF = B * F

        def do_round(r, _):
            m = sc * rounds + r

            plsc.subcore_barrier()

            @pl.when(sub == 0)
            def _stage_x():
                pltpu.sync_copy(xp.at[pl.ds(m * V, V)], stg)
                pltpu.sync_copy(stg, xs)

            @pl.when(sub == 1)
            def _stage_y():
                pltpu.sync_copy(yp.at[pl.ds(m * V, V)], stg)
                pltpu.sync_copy(stg, ys)

            @pl.when(sub == 2)
            def _stage_z():
                pltpu.sync_copy(zp.at[pl.ds(m * V, V)], stg)
                pltpu.sync_copy(stg, zs)

            plsc.subcore_barrier()

            def do_chunk(j, _):
                k = j * _NSUB + sub

                @pl.when(k < n_chunks)
                def _chunk():
                    p0 = m * F + k * C
                    pltpu.sync_copy(fc_hbm.at[pl.ds(p0, C)], i0)
                    pltpu.sync_copy(fc_hbm.at[pl.ds(BF + p0, C)], i1)
                    pltpu.sync_copy(fc_hbm.at[pl.ds(2 * BF + p0, C)], i2)

                    cps = [
                        pltpu.async_copy(xs.at[i0], x0b, sem),
                        pltpu.async_copy(ys.at[i0], y0b, sem),
                        pltpu.async_copy(zs.at[i0], z0b, sem),
                        pltpu.async_copy(xs.at[i1], x1b, sem),
                        pltpu.async_copy(ys.at[i1], y1b, sem),
                        pltpu.async_copy(zs.at[i1], z1b, sem),
                        pltpu.async_copy(xs.at[i2], x2b, sem),
                        pltpu.async_copy(ys.at[i2], y2b, sem),
                        pltpu.async_copy(zs.at[i2], z2b, sem),
                    ]
                    for cp in cps:
                        cp.wait()

                    def group(g, _):
                        sl = pl.ds(g * _L, _L)
                        x0 = x0b[sl]
                        y0 = y0b[sl]
                        z0 = z0b[sl]
                        x1 = x1b[sl]
                        y1 = y1b[sl]
                        z1 = z1b[sl]
                        x2 = x2b[sl]
                        y2 = y2b[sl]
                        z2 = z2b[sl]
                        e1x = x0 - x1
                        e1y = y0 - y1
                        e1z = z0 - z1
                        e2x = x2 - x1
                        e2y = y2 - y1
                        e2z = z2 - z1
                        nx = e2y * e1z - e2z * e1y
                        ny = e2z * e1x - e2x * e1z
                        nz = e2x * e1y - e2y * e1x
                        s = jnp.maximum(nx * nx + ny * ny + nz * nz, 1e-24)
                        t = plsc.bitcast(s, jnp.int32)
                        t = 0x5F3759DF - lax.shift_right_logical(t, 1)
                        y = plsc.bitcast(t, jnp.float32)
                        hs = 0.5 * s
                        y = y * (1.5 - hs * y * y)
                        y = y * (1.5 - hs * y * y)
                        y = y * (1.5 - hs * y * y)
                        oxb[sl] = nx * y
                        oyb[sl] = ny * y
                        ozb[sl] = nz * y
                        return 0

                    lax.fori_loop(0, groups, group, 0)

                    pltpu.sync_copy(oxb, out_hbm.at[pl.ds(p0, C)])
                    pltpu.sync_copy(oyb, out_hbm.at[pl.ds(BF + p0, C)])
                    pltpu.sync_copy(ozb, out_hbm.at[pl.ds(2 * BF + p0, C)])

                return 0

            lax.fori_loop(0, chunk_iters, do_chunk, 0)
            return 0

        lax.fori_loop(0, rounds, do_round, 0)

    return body


def kernel(vertices, faces):
    B, V, _ = vertices.shape
    _, F, _ = faces.shape
    vtt = jnp.transpose(vertices, (2, 0, 1))
    xp = vtt[0].reshape(B * V)
    yp = vtt[1].reshape(B * V)
    zp = vtt[2].reshape(B * V)
    fc = jnp.transpose(faces, (2, 0, 1)).reshape(3 * B * F)
    out = _face_normals_impl(B, V, F)(xp, yp, zp, fc)
    return jnp.transpose(out.reshape(3, B, F), (1, 2, 0))
